# Initial kernel scaffold; baseline (speedup 1.0000x reference)
import jax
import jax.numpy as jnp
from jax.experimental import pallas as pl


def kernel(weights, codebook):
    raise NotImplementedError("write your pallas kernel here")


# speedup vs baseline: 1.3500x; 1.3500x over previous
import functools

import jax
import jax.numpy as jnp
from jax import lax
from jax.experimental import pallas as pl
from jax.experimental.pallas import tpu as pltpu
from jax.experimental.pallas import tpu_sc as plsc


_R = 4 * 1024
_K = 8192
_BR = 128


def _argmax_body(w_ref, out_ref):
    w = w_ref[...]
    m = jnp.max(w, axis=-1, keepdims=True)
    ii = lax.broadcasted_iota(jnp.int32, w.shape, 1)
    idx = jnp.min(jnp.where(w == m, ii, _K), axis=-1)
    out_ref[0, 0, :] = idx


def _argmax_rows(w2):
    grid = _R // _BR
    return pl.pallas_call(
        _argmax_body,
        grid=(grid,),
        in_specs=[pl.BlockSpec((_BR, _K), lambda i: (i, 0))],
        out_specs=pl.BlockSpec((1, 1, _BR), lambda i: (i, 0, 0)),
        out_shape=jax.ShapeDtypeStruct((grid, 1, _BR), jnp.int32),
    )(w2)


_NC, _NS = 2, 16
_NW = _NC * _NS
_BPW = _R // _NW
_D = 256

_sc_mesh = plsc.VectorSubcoreMesh(core_axis_name="c", subcore_axis_name="s")


@functools.partial(
    pl.kernel,
    mesh=_sc_mesh,
    out_type=jax.ShapeDtypeStruct((_R, _D), jnp.float32),
    scratch_types=[
        pltpu.VMEM((_BPW,), jnp.int32),
        pltpu.VMEM((_BPW, _D), jnp.float32),
        pltpu.SemaphoreType.DMA,
    ],
)
def _sc_gather(table_hbm, idx_hbm, out_hbm, idx_v, rows_v, sem):
    wid = lax.axis_index("s") * _NC + lax.axis_index("c")
    base = wid * _BPW
    pltpu.sync_copy(idx_hbm.at[pl.ds(base, _BPW)], idx_v)
    pltpu.async_copy(table_hbm.at[idx_v], rows_v, sem).wait()
    pltpu.sync_copy(rows_v, out_hbm.at[pl.ds(base, _BPW)])


@jax.jit
def kernel(weights, codebook):
    w2 = weights.reshape(_R, _K)
    indices = _argmax_rows(w2).reshape(_R)
    rows = _sc_gather(codebook, indices)
    return rows.reshape(4, 1024, _D)
